# Initial kernel scaffold; baseline (speedup 1.0000x reference)
import jax
import jax.numpy as jnp
from jax.experimental import pallas as pl


def kernel(x, t, batch_idx, gen_flag, noise, alphas_cumprod):
    raise NotImplementedError("write your pallas kernel here")


# speedup vs baseline: 6.8270x; 6.8270x over previous
import functools

import jax
import jax.numpy as jnp
from jax import lax
from jax.experimental import pallas as pl
from jax.experimental.pallas import tpu as pltpu
from jax.experimental.pallas import tpu_sc as plsc

_NC = 2
_NS = 16
_NW = _NC * _NS
_L = 16


def _newton_sqrt(a):
    bits = plsc.bitcast(a, jnp.int32)
    y = plsc.bitcast(jnp.int32(0x5F3759DF) - (bits >> 1), jnp.float32)
    for _ in range(3):
        y = y * (1.5 - 0.5 * a * y * y)
    return a * y


def _sc_coeffs(ac_pad, t, batch_idx, flag_i32, n_nodes, num_b, chunk):
    per_tile = n_nodes // _NW
    n_chunks = per_tile // chunk
    tbl_iters = num_b // _L
    node_iters = chunk // _L

    mesh = plsc.VectorSubcoreMesh(core_axis_name="c", subcore_axis_name="s")

    @functools.partial(
        pl.kernel,
        mesh=mesh,
        compiler_params=pltpu.CompilerParams(needs_layout_passes=False),
        out_type=[
            jax.ShapeDtypeStruct((n_nodes,), jnp.float32),
            jax.ShapeDtypeStruct((n_nodes,), jnp.float32),
        ],
        scratch_types=[
            pltpu.VMEM((ac_pad.shape[0],), jnp.float32),
            pltpu.VMEM((num_b,), jnp.int32),
            pltpu.VMEM((num_b,), jnp.float32),
            pltpu.VMEM((num_b,), jnp.float32),
            pltpu.VMEM((chunk,), jnp.int32),
            pltpu.VMEM((chunk,), jnp.int32),
            pltpu.VMEM((chunk,), jnp.float32),
            pltpu.VMEM((chunk,), jnp.float32),
        ],
    )
    def sc_k(ac_hbm, t_hbm, bidx_hbm, flag_hbm, sa_hbm, sb_hbm,
             ac_v, t_v, sa_v, sb_v, bi_v, fl_v, sao_v, sbo_v):
        wid = lax.axis_index("s") * _NC + lax.axis_index("c")
        pltpu.sync_copy(ac_hbm, ac_v)
        pltpu.sync_copy(t_hbm, t_v)

        def table_body(k, carry):
            sl = pl.ds(k * _L, _L)
            av = plsc.load_gather(ac_v, [t_v[sl]])
            sa_v[sl] = _newton_sqrt(av)
            sb_v[sl] = _newton_sqrt(1.0 - av)
            return carry

        lax.fori_loop(0, tbl_iters, table_body, 0)

        base = wid * per_tile
        for c in range(n_chunks):
            off = base + c * chunk
            pltpu.sync_copy(bidx_hbm.at[pl.ds(off, chunk)], bi_v)
            pltpu.sync_copy(flag_hbm.at[pl.ds(off, chunk)], fl_v)

            def node_body(i, carry):
                sl = pl.ds(i * _L, _L)
                bv = bi_v[sl]
                ok = fl_v[sl] != 0
                sao_v[sl] = jnp.where(ok, plsc.load_gather(sa_v, [bv]), 1.0)
                sbo_v[sl] = jnp.where(ok, plsc.load_gather(sb_v, [bv]), 0.0)
                return carry

            lax.fori_loop(0, node_iters, node_body, 0)
            pltpu.sync_copy(sao_v, sa_hbm.at[pl.ds(off, chunk)])
            pltpu.sync_copy(sbo_v, sb_hbm.at[pl.ds(off, chunk)])

    return sc_k(ac_pad, t, batch_idx, flag_i32)


def _tc_body(x_ref, n_ref, sa_ref, sb_ref, o_ref):
    lane = lax.broadcasted_iota(jnp.int32, (8, 128), 1)
    row = lax.broadcasted_iota(jnp.int32, (8, 128), 0)
    kf = ((lane // 16) == row).astype(jnp.float32)
    dot = functools.partial(
        jnp.dot, precision=lax.Precision.HIGHEST,
        preferred_element_type=jnp.float32)
    sa = dot(sa_ref[...], kf)
    sb = dot(sb_ref[...], kf)
    o_ref[...] = sa * x_ref[...] + sb * n_ref[...]


def kernel(x, t, batch_idx, gen_flag, noise, alphas_cumprod):
    n, d = x.shape
    num_b = t.shape[0]
    num_t = alphas_cumprod.shape[0]

    pad = (-num_t) % 16
    ac_pad = jnp.concatenate(
        [alphas_cumprod, jnp.zeros((pad,), jnp.float32)]) if pad else alphas_cumprod
    flag_i32 = gen_flag.astype(jnp.int32)

    sa_n, sb_n = _sc_coeffs(ac_pad, t, batch_idx, flag_i32,
                            n_nodes=n, num_b=num_b, chunk=8192)

    lanes = 128
    nodes_per_row = lanes // d
    nr = n // nodes_per_row
    rblk = 2048
    x2 = x.reshape(nr, lanes)
    n2 = noise.reshape(nr, lanes)
    sa8 = sa_n.reshape(nr, nodes_per_row)
    sb8 = sb_n.reshape(nr, nodes_per_row)

    out2 = pl.pallas_call(
        _tc_body,
        grid=(nr // rblk,),
        in_specs=[
            pl.BlockSpec((rblk, lanes), lambda i: (i, 0)),
            pl.BlockSpec((rblk, lanes), lambda i: (i, 0)),
            pl.BlockSpec((rblk, nodes_per_row), lambda i: (i, 0)),
            pl.BlockSpec((rblk, nodes_per_row), lambda i: (i, 0)),
        ],
        out_specs=pl.BlockSpec((rblk, lanes), lambda i: (i, 0)),
        out_shape=jax.ShapeDtypeStruct((nr, lanes), jnp.float32),
    )(x2, n2, sa8, sb8)

    return (out2.reshape(n, d), noise)
